# baseline (speedup 1.0000x reference)
import functools

import jax
import jax.numpy as jnp
from jax import lax
from jax.experimental import pallas as pl
from jax.experimental.pallas import tpu as pltpu
from jax.experimental.pallas import tpu_sc as plsc

H = 2048
E = 64
K = 8
L = 16
_NEG = -1e30


def _gate_body(x_ref, wn_ref, wt_ref, noise_ref, mw_ref, acc_sp, acc_x,
               *, n_tokens):
    i = pl.program_id(0)

    @pl.when(i == 0)
    def _init():
        acc_sp[...] = jnp.zeros_like(acc_sp)
        acc_x[...] = jnp.zeros_like(acc_x)

    xb = x_ref[...]
    g = lax.dot_general(xb, wn_ref[...], (((1,), (1,)), ((), ())),
                        preferred_element_type=jnp.float32)
    acc_sp[...] += jnp.sum(jax.nn.softplus(g), axis=0, keepdims=True)
    acc_x[...] += jnp.sum(xb, axis=0, keepdims=True)

    @pl.when(i == pl.num_programs(0) - 1)
    def _finish():
        ninv = jnp.float32(1.0 / n_tokens)
        mean_t = lax.dot_general(acc_x[...] * ninv, wt_ref[...],
                                 (((1,), (1,)), ((), ())),
                                 preferred_element_type=jnp.float32)
        mw_ref[...] = mean_t + acc_sp[...] * ninv * noise_ref[...]


def _topk_sc_body(mw_hbm, gates_hbm, idx_hbm, mw_v, g_v, i_v):
    cid = lax.axis_index("c")
    sid = lax.axis_index("s")

    @pl.when((cid == 0) & (sid == 0))
    def _():
        pltpu.sync_copy(mw_hbm, mw_v)
        lane = lax.iota(jnp.int32, L)
        vals = [mw_v[pl.ds(j * L, L)] for j in range(E // L)]
        iotas = [lane + j * L for j in range(E // L)]
        gsel = jnp.zeros((L,), jnp.float32)
        isel = jnp.zeros((L,), jnp.int32)
        g0 = jnp.float32(0.0)
        for k in range(K):
            mx = vals[0]
            for j in range(1, E // L):
                mx = jnp.maximum(mx, vals[j])
            m = jnp.max(mx)
            if k == 0:
                g0 = m
            cand = jnp.where(vals[0] == m, iotas[0], E)
            for j in range(1, E // L):
                cand = jnp.minimum(cand, jnp.where(vals[j] == m, iotas[j], E))
            sel = jnp.min(cand)
            gsel = jnp.where(lane == k, m, gsel)
            isel = jnp.where(lane == k, sel, isel)
            for j in range(E // L):
                vals[j] = jnp.where(iotas[j] == sel, _NEG, vals[j])
        e = jnp.where(lane < K, jnp.exp(gsel - g0), 0.0)
        s = jnp.sum(e)
        g_v[...] = e / s
        i_v[...] = isel
        pltpu.sync_copy(g_v, gates_hbm)
        pltpu.sync_copy(i_v, idx_hbm)


_topk_sc = pl.kernel(
    _topk_sc_body,
    out_type=(
        jax.ShapeDtypeStruct((L,), jnp.float32),
        jax.ShapeDtypeStruct((L,), jnp.int32),
    ),
    mesh=plsc.VectorSubcoreMesh(core_axis_name="c", subcore_axis_name="s"),
    scratch_types=[
        pltpu.VMEM((E,), jnp.float32),
        pltpu.VMEM((L,), jnp.float32),
        pltpu.VMEM((L,), jnp.int32),
    ],
    compiler_params=pltpu.CompilerParams(needs_layout_passes=False),
)


def kernel(x, W_transform, W_noise):
    n_tokens = x.shape[0] * x.shape[1]
    x2d = x.reshape(n_tokens, H)
    noise = jax.random.normal(jax.random.key(42), (E,), dtype=x.dtype)
    noise2d = noise.reshape(1, E)

    bt = 1024
    grid = (n_tokens // bt,)
    mw = pl.pallas_call(
        functools.partial(_gate_body, n_tokens=n_tokens),
        grid=grid,
        in_specs=[
            pl.BlockSpec((bt, H), lambda i: (i, 0)),
            pl.BlockSpec((E, H), lambda i: (0, 0)),
            pl.BlockSpec((E, H), lambda i: (0, 0)),
            pl.BlockSpec((1, E), lambda i: (0, 0)),
        ],
        out_specs=pl.BlockSpec((1, E), lambda i: (0, 0)),
        out_shape=jax.ShapeDtypeStruct((1, E), jnp.float32),
        scratch_shapes=[
            pltpu.VMEM((1, E), jnp.float32),
            pltpu.VMEM((1, H), jnp.float32),
        ],
    )(x2d, W_noise, W_transform, noise2d)

    gates16, idx16 = _topk_sc(mw.reshape(E))
    return gates16[:K], idx16[:K]


# speedup vs baseline: 1.0184x; 1.0184x over previous
import functools

import jax
import jax.numpy as jnp
from jax import lax
from jax.experimental import pallas as pl
from jax.experimental.pallas import tpu as pltpu
from jax.experimental.pallas import tpu_sc as plsc

H = 2048
E = 64
K = 8
L = 16
_NEG = -1e30


def _gate_body(x_ref, wn_ref, wt_ref, noise_ref, mw_ref, acc_sp, acc_x,
               *, n_tokens):
    i = pl.program_id(0)

    @pl.when(i == 0)
    def _init():
        acc_sp[...] = jnp.zeros_like(acc_sp)
        acc_x[...] = jnp.zeros_like(acc_x)

    xb = x_ref[...]
    g = lax.dot_general(xb, wn_ref[...], (((1,), (1,)), ((), ())),
                        preferred_element_type=jnp.float32)
    acc_sp[...] += jnp.sum(jax.nn.softplus(g), axis=0, keepdims=True)
    acc_x[...] += jnp.sum(xb, axis=0, keepdims=True)

    @pl.when(i == pl.num_programs(0) - 1)
    def _finish():
        ninv = jnp.float32(1.0 / n_tokens)
        mean_t = lax.dot_general(acc_x[...] * ninv, wt_ref[...],
                                 (((1,), (1,)), ((), ())),
                                 preferred_element_type=jnp.float32)
        mw_ref[...] = mean_t + acc_sp[...] * ninv * noise_ref[...]


def _topk_sc_body(mw_hbm, gates_hbm, idx_hbm, mw_v, g_v, i_v):
    cid = lax.axis_index("c")
    sid = lax.axis_index("s")

    @pl.when((cid == 0) & (sid == 0))
    def _():
        pltpu.sync_copy(mw_hbm, mw_v)
        lane = lax.iota(jnp.int32, L)
        vals = [mw_v[pl.ds(j * L, L)] for j in range(E // L)]
        iotas = [lane + j * L for j in range(E // L)]
        gsel = jnp.zeros((L,), jnp.float32)
        isel = jnp.zeros((L,), jnp.int32)
        g0 = jnp.float32(0.0)
        for k in range(K):
            mx = vals[0]
            for j in range(1, E // L):
                mx = jnp.maximum(mx, vals[j])
            m = jnp.max(mx)
            if k == 0:
                g0 = m
            cand = jnp.where(vals[0] == m, iotas[0], E)
            for j in range(1, E // L):
                cand = jnp.minimum(cand, jnp.where(vals[j] == m, iotas[j], E))
            sel = jnp.min(cand)
            gsel = jnp.where(lane == k, m, gsel)
            isel = jnp.where(lane == k, sel, isel)
            for j in range(E // L):
                vals[j] = jnp.where(iotas[j] == sel, _NEG, vals[j])
        e = jnp.where(lane < K, jnp.exp(gsel - g0), 0.0)
        s = jnp.sum(e)
        g_v[...] = e / s
        i_v[...] = isel
        pltpu.sync_copy(g_v, gates_hbm)
        pltpu.sync_copy(i_v, idx_hbm)


_topk_sc = pl.kernel(
    _topk_sc_body,
    out_type=(
        jax.ShapeDtypeStruct((L,), jnp.float32),
        jax.ShapeDtypeStruct((L,), jnp.int32),
    ),
    mesh=plsc.VectorSubcoreMesh(core_axis_name="c", subcore_axis_name="s",
                                num_cores=1),
    scratch_types=[
        pltpu.VMEM((E,), jnp.float32),
        pltpu.VMEM((L,), jnp.float32),
        pltpu.VMEM((L,), jnp.int32),
    ],
    compiler_params=pltpu.CompilerParams(needs_layout_passes=False),
)


def kernel(x, W_transform, W_noise):
    n_tokens = x.shape[0] * x.shape[1]
    x2d = x.reshape(n_tokens, H)
    noise = jax.random.normal(jax.random.key(42), (E,), dtype=x.dtype)
    noise2d = noise.reshape(1, E)

    bt = 1024
    grid = (n_tokens // bt,)
    mw = pl.pallas_call(
        functools.partial(_gate_body, n_tokens=n_tokens),
        grid=grid,
        in_specs=[
            pl.BlockSpec((bt, H), lambda i: (i, 0)),
            pl.BlockSpec((E, H), lambda i: (0, 0)),
            pl.BlockSpec((E, H), lambda i: (0, 0)),
            pl.BlockSpec((1, E), lambda i: (0, 0)),
        ],
        out_specs=pl.BlockSpec((1, E), lambda i: (0, 0)),
        out_shape=jax.ShapeDtypeStruct((1, E), jnp.float32),
        scratch_shapes=[
            pltpu.VMEM((1, E), jnp.float32),
            pltpu.VMEM((1, H), jnp.float32),
        ],
    )(x2d, W_noise, W_transform, noise2d)

    gates16, idx16 = _topk_sc(mw.reshape(E))
    return gates16[:K], idx16[:K]


# speedup vs baseline: 1.0185x; 1.0000x over previous
import functools

import jax
import jax.numpy as jnp
from jax import lax
from jax.experimental import pallas as pl
from jax.experimental.pallas import tpu as pltpu
from jax.experimental.pallas import tpu_sc as plsc

H = 2048
E = 64
K = 8
L = 16
_NEG = -1e30


def _gate_body(x_ref, wn_ref, wt_ref, noise_ref, mw_ref, acc_sp, acc_x,
               *, n_tokens):
    i = pl.program_id(0)

    @pl.when(i == 0)
    def _init():
        acc_sp[...] = jnp.zeros_like(acc_sp)
        acc_x[...] = jnp.zeros_like(acc_x)

    xb = x_ref[...]
    g = lax.dot_general(xb, wn_ref[...], (((1,), (1,)), ((), ())),
                        preferred_element_type=jnp.float32)
    acc_sp[...] += jnp.sum(jax.nn.softplus(g), axis=0, keepdims=True)
    acc_x[...] += jnp.sum(xb, axis=0, keepdims=True)

    @pl.when(i == pl.num_programs(0) - 1)
    def _finish():
        ninv = jnp.float32(1.0 / n_tokens)
        mean_t = lax.dot_general(acc_x[...] * ninv, wt_ref[...],
                                 (((1,), (1,)), ((), ())),
                                 preferred_element_type=jnp.float32)
        mw_ref[...] = mean_t + acc_sp[...] * ninv * noise_ref[...]


def _topk_sc_body(mw_hbm, gates_hbm, idx_hbm, mw_v, g_v, i_v):
    cid = lax.axis_index("c")
    sid = lax.axis_index("s")

    @pl.when((cid == 0) & (sid == 0))
    def _():
        pltpu.sync_copy(mw_hbm, mw_v)
        lane = lax.iota(jnp.int32, L)
        vals = [mw_v[pl.ds(j * L, L)] for j in range(E // L)]
        iotas = [lane + j * L for j in range(E // L)]
        gsel = jnp.zeros((L,), jnp.float32)
        isel = jnp.zeros((L,), jnp.int32)
        g0 = jnp.float32(0.0)
        for k in range(K):
            mx = vals[0]
            for j in range(1, E // L):
                mx = jnp.maximum(mx, vals[j])
            m = jnp.max(mx)
            if k == 0:
                g0 = m
            cand = jnp.where(vals[0] == m, iotas[0], E)
            for j in range(1, E // L):
                cand = jnp.minimum(cand, jnp.where(vals[j] == m, iotas[j], E))
            sel = jnp.min(cand)
            gsel = jnp.where(lane == k, m, gsel)
            isel = jnp.where(lane == k, sel, isel)
            for j in range(E // L):
                vals[j] = jnp.where(iotas[j] == sel, _NEG, vals[j])
        e = jnp.where(lane < K, jnp.exp(gsel - g0), 0.0)
        s = jnp.sum(e)
        g_v[...] = e / s
        i_v[...] = isel
        pltpu.sync_copy(g_v, gates_hbm)
        pltpu.sync_copy(i_v, idx_hbm)


_topk_sc = pl.kernel(
    _topk_sc_body,
    out_type=(
        jax.ShapeDtypeStruct((L,), jnp.float32),
        jax.ShapeDtypeStruct((L,), jnp.int32),
    ),
    mesh=plsc.VectorSubcoreMesh(core_axis_name="c", subcore_axis_name="s",
                                num_cores=1),
    scratch_types=[
        pltpu.VMEM((E,), jnp.float32),
        pltpu.VMEM((L,), jnp.float32),
        pltpu.VMEM((L,), jnp.int32),
    ],
    compiler_params=pltpu.CompilerParams(needs_layout_passes=False,
                                         skip_device_barrier=True),
)


def kernel(x, W_transform, W_noise):
    n_tokens = x.shape[0] * x.shape[1]
    x2d = x.reshape(n_tokens, H)
    noise = jax.random.normal(jax.random.key(42), (E,), dtype=x.dtype)
    noise2d = noise.reshape(1, E)

    bt = 1024
    grid = (n_tokens // bt,)
    mw = pl.pallas_call(
        functools.partial(_gate_body, n_tokens=n_tokens),
        grid=grid,
        in_specs=[
            pl.BlockSpec((bt, H), lambda i: (i, 0)),
            pl.BlockSpec((E, H), lambda i: (0, 0)),
            pl.BlockSpec((E, H), lambda i: (0, 0)),
            pl.BlockSpec((1, E), lambda i: (0, 0)),
        ],
        out_specs=pl.BlockSpec((1, E), lambda i: (0, 0)),
        out_shape=jax.ShapeDtypeStruct((1, E), jnp.float32),
        scratch_shapes=[
            pltpu.VMEM((1, E), jnp.float32),
            pltpu.VMEM((1, H), jnp.float32),
        ],
    )(x2d, W_noise, W_transform, noise2d)

    gates16, idx16 = _topk_sc(mw.reshape(E))
    return gates16[:K], idx16[:K]
